# Initial kernel scaffold; baseline (speedup 1.0000x reference)
import jax
import jax.numpy as jnp
from jax.experimental import pallas as pl


def kernel(x_empresa, x_socio, ei_owns, ei_owned_by, ei_related, W_lin_e, W_lin_s, Wl1_owns, Wr1_owns, Wl1_ownedby, Wr1_ownedby, Wl1_rel, Wr1_rel, Wl2_owns, Wr2_owns, Wl2_ownedby, Wr2_ownedby, Ws2, Wd2, b_lin_e, b_lin_s, b1_owns, b1_ownedby, b1_rel, b2_owns, b2_ownedby, as2, ad2, bg2):
    raise NotImplementedError("write your pallas kernel here")


# speedup vs baseline: 7.0045x; 7.0045x over previous
import functools

import jax
import jax.numpy as jnp
from jax import lax
from jax.experimental import pallas as pl
from jax.experimental.pallas import tpu as pltpu
from jax.experimental.pallas import tpu_sc as plsc

N_E = 10000
N_S = 10000
DH = 128
EDG = 320000

NPAD = 10240
NW = 32
EPW = EDG // NW
CH = 80
NCHUNK = EPW // CH
RPS = NPAD // 16


def _splat(v, l):
    idx = jnp.full((16, 1), l, jnp.int32)
    return lax.gather(
        v, idx,
        lax.GatherDimensionNumbers(
            offset_dims=(), collapsed_slice_dims=(0,), start_index_map=(0,)),
        (1,), mode=lax.GatherScatterMode.PROMISE_IN_BOUNDS)


def _sc_agg_layer1(ei_owns, ei_ob, ei_rel, xe, xs, zr2, zr1):
    mesh = plsc.VectorSubcoreMesh(core_axis_name="c", subcore_axis_name="s")
    out_type = [
        jax.ShapeDtypeStruct((2, NPAD, DH), jnp.float32),
        jax.ShapeDtypeStruct((2, NPAD, DH), jnp.float32),
        jax.ShapeDtypeStruct((2, NPAD, DH), jnp.float32),
        jax.ShapeDtypeStruct((2, NPAD), jnp.float32),
        jax.ShapeDtypeStruct((2, NPAD), jnp.float32),
        jax.ShapeDtypeStruct((2, NPAD), jnp.float32),
    ]
    scratch = [
        pltpu.VMEM((CH,), jnp.int32),
        pltpu.VMEM((CH,), jnp.int32),
        pltpu.VMEM((CH, DH), jnp.float32),
        pltpu.VMEM((CH,), jnp.float32),
        pltpu.VMEM_SHARED((NPAD, DH), jnp.float32),
        pltpu.VMEM_SHARED((NPAD,), jnp.float32),
        pltpu.SemaphoreType.DMA,
    ]

    @functools.partial(pl.kernel, mesh=mesh, out_type=out_type,
                       scratch_types=scratch)
    def k(owns_s, owns_d, ob_s, ob_d, rel_s, rel_d, xe_ref, xs_ref,
          zr2_ref, zr1_ref,
          s1_ref, s2_ref, s3_ref, c1_ref, c2_ref, c3_ref,
          sidx, didx, rows, ones, acc, cacc, sem):
        cid = lax.axis_index("c")
        sid = lax.axis_index("s")
        gw = cid * 16 + sid

        for j in range(CH // 16):
            ones[pl.ds(j * 16, 16)] = jnp.full((16,), 1.0, jnp.float32)

        def seg_phase(es_ref, ed_ref, x_ref, out_ref, cnt_ref):
            pltpu.sync_copy(zr2_ref, acc.at[pl.ds(sid * RPS, RPS)])
            pltpu.sync_copy(zr1_ref, cacc.at[pl.ds(sid * RPS, RPS)])
            plsc.subcore_barrier()

            def body(i, carry):
                base = pl.multiple_of(gw * EPW + i * CH, 8)
                pltpu.sync_copy(es_ref.at[pl.ds(base, CH)], sidx)
                pltpu.sync_copy(ed_ref.at[pl.ds(base, CH)], didx)
                pltpu.async_copy(x_ref.at[sidx], rows, sem).wait()
                pltpu.sync_copy(rows, acc.at[didx], add=True)
                pltpu.sync_copy(ones, cacc.at[didx], add=True)
                return carry

            lax.fori_loop(0, NCHUNK, body, 0)
            plsc.subcore_barrier()
            pltpu.sync_copy(acc.at[pl.ds(sid * RPS, RPS)],
                            out_ref.at[cid, pl.ds(sid * RPS, RPS)])
            pltpu.sync_copy(cacc.at[pl.ds(sid * RPS, RPS)],
                            cnt_ref.at[cid, pl.ds(sid * RPS, RPS)])
            plsc.subcore_barrier()

        seg_phase(owns_s, owns_d, xe_ref, s1_ref, c1_ref)
        seg_phase(ob_s, ob_d, xs_ref, s2_ref, c2_ref)
        seg_phase(rel_s, rel_d, xe_ref, s3_ref, c3_ref)

    return k(ei_owns[0], ei_owns[1], ei_ob[0], ei_ob[1], ei_rel[0],
             ei_rel[1], xe, xs, zr2, zr1)


def _sc_agg_layer2(ei_owns, ei_ob, ei_rel, he, hs, hsg, sarr, darr, smax16,
                   zr2, zr1):
    mesh = plsc.VectorSubcoreMesh(core_axis_name="c", subcore_axis_name="s")
    out_type = [
        jax.ShapeDtypeStruct((2, NPAD, DH), jnp.float32),
        jax.ShapeDtypeStruct((2, NPAD, DH), jnp.float32),
        jax.ShapeDtypeStruct((2, NPAD, DH), jnp.float32),
        jax.ShapeDtypeStruct((2, NPAD), jnp.float32),
    ]
    scratch = [
        pltpu.VMEM((CH,), jnp.int32),
        pltpu.VMEM((CH,), jnp.int32),
        pltpu.VMEM((CH, DH), jnp.float32),
        pltpu.VMEM((CH,), jnp.float32),
        pltpu.VMEM((CH,), jnp.float32),
        pltpu.VMEM((CH,), jnp.float32),
        pltpu.VMEM((16,), jnp.float32),
        pltpu.VMEM_SHARED((NPAD, DH), jnp.float32),
        pltpu.VMEM_SHARED((NPAD,), jnp.float32),
        pltpu.SemaphoreType.DMA,
    ]

    @functools.partial(pl.kernel, mesh=mesh, out_type=out_type,
                       scratch_types=scratch)
    def k(owns_s, owns_d, ob_s, ob_d, rel_s, rel_d, he_ref, hs_ref,
          hsg_ref, s_ref, d_ref, smax_ref, zr2_ref, zr1_ref,
          s4_ref, s5_ref, num_ref, den_ref,
          sidx, didx, rows, sval, dval, wvec, smv, acc, dacc, sem):
        cid = lax.axis_index("c")
        sid = lax.axis_index("s")
        gw = cid * 16 + sid

        pltpu.sync_copy(smax_ref, smv)

        def seg_phase(es_ref, ed_ref, x_ref, out_ref):
            pltpu.sync_copy(zr2_ref, acc.at[pl.ds(sid * RPS, RPS)])
            plsc.subcore_barrier()

            def body(i, carry):
                base = pl.multiple_of(gw * EPW + i * CH, 8)
                pltpu.sync_copy(es_ref.at[pl.ds(base, CH)], sidx)
                pltpu.sync_copy(ed_ref.at[pl.ds(base, CH)], didx)
                pltpu.async_copy(x_ref.at[sidx], rows, sem).wait()
                pltpu.sync_copy(rows, acc.at[didx], add=True)
                return carry

            lax.fori_loop(0, NCHUNK, body, 0)
            plsc.subcore_barrier()
            pltpu.sync_copy(acc.at[pl.ds(sid * RPS, RPS)],
                            out_ref.at[cid, pl.ds(sid * RPS, RPS)])
            plsc.subcore_barrier()

        seg_phase(owns_s, owns_d, he_ref, s4_ref)
        seg_phase(ob_s, ob_d, hs_ref, s5_ref)

        pltpu.sync_copy(zr2_ref, acc.at[pl.ds(sid * RPS, RPS)])
        pltpu.sync_copy(zr1_ref, dacc.at[pl.ds(sid * RPS, RPS)])
        plsc.subcore_barrier()

        def gat_body(i, carry):
            base = pl.multiple_of(gw * EPW + i * CH, 8)
            pltpu.sync_copy(rel_s.at[pl.ds(base, CH)], sidx)
            pltpu.sync_copy(rel_d.at[pl.ds(base, CH)], didx)
            pltpu.async_copy(s_ref.at[sidx], sval, sem).wait()
            pltpu.async_copy(d_ref.at[didx], dval, sem).wait()
            pltpu.async_copy(hsg_ref.at[sidx], rows, sem).wait()
            sm = smv[...]
            for g in range(CH // 16):
                sv = sval[pl.ds(g * 16, 16)]
                dv = dval[pl.ds(g * 16, 16)]
                e = sv + dv
                e = jnp.maximum(e, 0.2 * e)
                m = sm + dv
                m = jnp.maximum(m, 0.2 * m)
                w = jnp.exp(e - m)
                wvec[pl.ds(g * 16, 16)] = w
                for l in range(16):
                    wsp = _splat(w, l)
                    r = g * 16 + l
                    for c in range(DH // 16):
                        rows[r, pl.ds(c * 16, 16)] = (
                            rows[r, pl.ds(c * 16, 16)] * wsp)
            pltpu.sync_copy(wvec, dacc.at[didx], add=True)
            pltpu.sync_copy(rows, acc.at[didx], add=True)
            return carry

        lax.fori_loop(0, NCHUNK, gat_body, 0)
        plsc.subcore_barrier()
        pltpu.sync_copy(acc.at[pl.ds(sid * RPS, RPS)],
                        num_ref.at[cid, pl.ds(sid * RPS, RPS)])
        pltpu.sync_copy(dacc.at[pl.ds(sid * RPS, RPS)],
                        den_ref.at[cid, pl.ds(sid * RPS, RPS)])

    return k(ei_owns[0], ei_owns[1], ei_ob[0], ei_ob[1], ei_rel[0],
             ei_rel[1], he, hs, hsg, sarr, darr, smax16, zr2, zr1)


_R = 512
_GRID = NPAD // _R


def _row_spec():
    return pl.BlockSpec((_R, DH), lambda i: (i, 0))


def _w_spec():
    return pl.BlockSpec((DH, DH), lambda i: (0, 0))


def _b_spec():
    return pl.BlockSpec((1, DH), lambda i: (0, 0))


def _p3_spec():
    return pl.BlockSpec((2, _R, DH), lambda i: (0, i, 0))


def _p2_spec():
    return pl.BlockSpec((2, _R, 1), lambda i: (0, i, 0))


def _stage_a(x_e, x_s, W_e, W_s, b_e, b_s):
    def body(xe_ref, xs_ref, we_ref, ws_ref, be_ref, bs_ref, oe_ref, os_ref):
        oe_ref[...] = jax.nn.relu(
            jnp.dot(xe_ref[...], we_ref[...],
                    preferred_element_type=jnp.float32) + be_ref[...])
        os_ref[...] = jax.nn.relu(
            jnp.dot(xs_ref[...], ws_ref[...],
                    preferred_element_type=jnp.float32) + bs_ref[...])

    return pl.pallas_call(
        body, grid=(_GRID,),
        in_specs=[_row_spec(), _row_spec(), _w_spec(), _w_spec(),
                  _b_spec(), _b_spec()],
        out_specs=[_row_spec(), _row_spec()],
        out_shape=[jax.ShapeDtypeStruct((NPAD, DH), jnp.float32)] * 2,
    )(x_e, x_s, W_e, W_s, b_e, b_s)


def _mean(p_ref, c_ref):
    s = p_ref[0] + p_ref[1]
    c = c_ref[0] + c_ref[1]
    return s / jnp.maximum(c, 1.0)


def _stage_c(s1p, c1p, s2p, c2p, s3p, c3p, xe, xs,
             Wl1o, Wr1o, b1o, Wl1b, Wr1b, b1b, Wl1r, Wr1r, b1r,
             Ws2, Wd2, as2c, ad2c):
    def body(s1_ref, c1_ref, s2_ref, c2_ref, s3_ref, c3_ref, xe_ref, xs_ref,
             wl1o_ref, wr1o_ref, b1o_ref, wl1b_ref, wr1b_ref, b1b_ref,
             wl1r_ref, wr1r_ref, b1r_ref, ws2_ref, wd2_ref, as2_ref, ad2_ref,
             hs_ref, he_ref, hsg_ref, s_ref, d_ref, smax_ref):
        i = pl.program_id(0)
        f32 = jnp.float32
        dot = lambda a, b: jnp.dot(a, b, preferred_element_type=f32)
        m1 = _mean(s1_ref, c1_ref)
        hs = jax.nn.relu(dot(m1, wl1o_ref[...]) +
                         dot(xs_ref[...], wr1o_ref[...]) + b1o_ref[...])
        m2 = _mean(s2_ref, c2_ref)
        m3 = _mean(s3_ref, c3_ref)
        he = jax.nn.relu(dot(m2, wl1b_ref[...]) +
                         dot(xe_ref[...], wr1b_ref[...]) + b1b_ref[...] +
                         dot(m3, wl1r_ref[...]) +
                         dot(xe_ref[...], wr1r_ref[...]) + b1r_ref[...])
        hsg = dot(he, ws2_ref[...])
        hdg = dot(he, wd2_ref[...])
        sc = dot(hsg, as2_ref[...])
        dc = dot(hdg, ad2_ref[...])
        hs_ref[...] = hs
        he_ref[...] = he
        hsg_ref[...] = hsg
        s_ref[...] = sc
        d_ref[...] = dc

        @pl.when(i == 0)
        def _():
            smax_ref[...] = jnp.full((1, 1), -jnp.inf, f32)

        smax_ref[...] = jnp.maximum(smax_ref[...], jnp.max(sc))

    col_spec = pl.BlockSpec((_R, 1), lambda i: (i, 0))
    return pl.pallas_call(
        body, grid=(_GRID,),
        in_specs=[_p3_spec(), _p2_spec(), _p3_spec(), _p2_spec(),
                  _p3_spec(), _p2_spec(), _row_spec(), _row_spec(),
                  _w_spec(), _w_spec(), _b_spec(), _w_spec(), _w_spec(),
                  _b_spec(), _w_spec(), _w_spec(), _b_spec(), _w_spec(),
                  _w_spec(), pl.BlockSpec((DH, 1), lambda i: (0, 0)),
                  pl.BlockSpec((DH, 1), lambda i: (0, 0))],
        out_specs=[_row_spec(), _row_spec(), _row_spec(), col_spec, col_spec,
                   pl.BlockSpec((1, 1), lambda i: (0, 0))],
        out_shape=[jax.ShapeDtypeStruct((NPAD, DH), jnp.float32),
                   jax.ShapeDtypeStruct((NPAD, DH), jnp.float32),
                   jax.ShapeDtypeStruct((NPAD, DH), jnp.float32),
                   jax.ShapeDtypeStruct((NPAD, 1), jnp.float32),
                   jax.ShapeDtypeStruct((NPAD, 1), jnp.float32),
                   jax.ShapeDtypeStruct((1, 1), jnp.float32)],
    )(s1p, c1p, s2p, c2p, s3p, c3p, xe, xs, Wl1o, Wr1o, b1o, Wl1b, Wr1b,
      b1b, Wl1r, Wr1r, b1r, Ws2, Wd2, as2c, ad2c)


def _stage_e(s4p, c1p, s5p, c2p, nump, denp, hs, he,
             Wl2o, Wr2o, b2o, Wl2b, Wr2b, b2b, bg2r):
    def body(s4_ref, c1_ref, s5_ref, c2_ref, num_ref, den_ref, hs_ref,
             he_ref, wl2o_ref, wr2o_ref, b2o_ref, wl2b_ref, wr2b_ref,
             b2b_ref, bg2_ref, oe_ref, os_ref):
        f32 = jnp.float32
        dot = lambda a, b: jnp.dot(a, b, preferred_element_type=f32)
        m4 = _mean(s4_ref, c1_ref)
        os_ref[...] = (dot(m4, wl2o_ref[...]) +
                       dot(hs_ref[...], wr2o_ref[...]) + b2o_ref[...])
        m5 = _mean(s5_ref, c2_ref)
        numer = num_ref[0] + num_ref[1]
        denom = den_ref[0] + den_ref[1]
        gat = numer / (denom + 1e-16)
        oe_ref[...] = (dot(m5, wl2b_ref[...]) +
                       dot(he_ref[...], wr2b_ref[...]) + b2b_ref[...] +
                       gat + bg2_ref[...])

    return pl.pallas_call(
        body, grid=(_GRID,),
        in_specs=[_p3_spec(), _p2_spec(), _p3_spec(), _p2_spec(),
                  _p3_spec(), _p2_spec(), _row_spec(), _row_spec(),
                  _w_spec(), _w_spec(), _b_spec(), _w_spec(), _w_spec(),
                  _b_spec(), _b_spec()],
        out_specs=[_row_spec(), _row_spec()],
        out_shape=[jax.ShapeDtypeStruct((NPAD, DH), jnp.float32)] * 2,
    )(s4p, c1p, s5p, c2p, nump, denp, hs, he, Wl2o, Wr2o, b2o, Wl2b,
      Wr2b, b2b, bg2r)


def kernel(x_empresa, x_socio, ei_owns, ei_owned_by, ei_related, W_lin_e,
           W_lin_s, Wl1_owns, Wr1_owns, Wl1_ownedby, Wr1_ownedby, Wl1_rel,
           Wr1_rel, Wl2_owns, Wr2_owns, Wl2_ownedby, Wr2_ownedby, Ws2, Wd2,
           b_lin_e, b_lin_s, b1_owns, b1_ownedby, b1_rel, b2_owns,
           b2_ownedby, as2, ad2, bg2):
    f32 = jnp.float32
    pad_e = NPAD - N_E
    xep = jnp.pad(x_empresa, ((0, pad_e), (0, 0)))
    xsp = jnp.pad(x_socio, ((0, NPAD - N_S), (0, 0)))
    eio = ei_owns.astype(jnp.int32)
    eib = ei_owned_by.astype(jnp.int32)
    eir = ei_related.astype(jnp.int32)
    zr2 = jnp.zeros((RPS, DH), f32)
    zr1 = jnp.zeros((RPS,), f32)

    xe, xs = _stage_a(xep, xsp, W_lin_e, W_lin_s,
                      b_lin_e.reshape(1, DH), b_lin_s.reshape(1, DH))

    s1p, s2p, s3p, c1p, c2p, c3p = _sc_agg_layer1(eio, eib, eir, xe, xs,
                                                  zr2, zr1)

    hs, he, hsg, sarr, darr, smax = _stage_c(
        s1p, c1p.reshape(2, NPAD, 1), s2p, c2p.reshape(2, NPAD, 1),
        s3p, c3p.reshape(2, NPAD, 1), xe, xs,
        Wl1_owns, Wr1_owns, b1_owns.reshape(1, DH),
        Wl1_ownedby, Wr1_ownedby, b1_ownedby.reshape(1, DH),
        Wl1_rel, Wr1_rel, b1_rel.reshape(1, DH),
        Ws2, Wd2, as2.reshape(DH, 1), ad2.reshape(DH, 1))

    smax16 = jnp.broadcast_to(smax.reshape(()), (16,)).astype(f32)
    s4p, s5p, nump, denp = _sc_agg_layer2(
        eio, eib, eir, he, hs, hsg,
        sarr.reshape(NPAD), darr.reshape(NPAD), smax16, zr2, zr1)

    oe, os_ = _stage_e(
        s4p, c1p.reshape(2, NPAD, 1), s5p, c2p.reshape(2, NPAD, 1),
        nump, denp.reshape(2, NPAD, 1), hs, he,
        Wl2_owns, Wr2_owns, b2_owns.reshape(1, DH),
        Wl2_ownedby, Wr2_ownedby, b2_ownedby.reshape(1, DH),
        bg2.reshape(1, DH))

    return (oe[:N_E], os_[:N_S])
